# Initial kernel scaffold; baseline (speedup 1.0000x reference)
import jax
import jax.numpy as jnp
from jax.experimental import pallas as pl


def kernel(x, edge_index, W1_0, W1_1, b1, W2_0, W2_1, b2):
    raise NotImplementedError("write your pallas kernel here")


# speedup vs baseline: 45.4140x; 45.4140x over previous
import functools

import jax
import jax.numpy as jnp
from jax import lax
from jax.experimental import pallas as pl
from jax.experimental.pallas import tpu as pltpu
from jax.experimental.pallas import tpu_sc as plsc

N = 10000
E = 320000
DF = 128
DH = 16
NPAD = 10112
NCORES = 2
NSUB = 16
NW = NCORES * NSUB
CHUNK = 128
GRP = 4
CPT = 80
EPAD = NW * CPT * CHUNK
RPT = NPAD // NSUB

_MESH = plsc.VectorSubcoreMesh(core_axis_name="c", subcore_axis_name="s")


@functools.partial(
    pl.kernel,
    mesh=_MESH,
    out_type=jax.ShapeDtypeStruct((NCORES, NPAD, DH), jnp.float32),
    scratch_types=[
        pltpu.VMEM((CPT, CHUNK), jnp.int32),
        pltpu.VMEM((CHUNK, DH), jnp.float32),
        pltpu.VMEM_SHARED((NPAD, DH), jnp.float32),
    ],
    compiler_params=pltpu.CompilerParams(use_tc_tiling_on_sc=False),
)
def _deg_sc(coli, ones_hbm, zrows, out, cv, onesv, acc):
    c = lax.axis_index("c")
    s = lax.axis_index("s")
    w = c * NSUB + s
    pltpu.sync_copy(zrows.at[pl.ds(s * RPT, RPT)], acc.at[pl.ds(s * RPT, RPT)])
    pltpu.sync_copy(coli.at[w], cv)
    pltpu.sync_copy(ones_hbm, onesv)
    plsc.subcore_barrier()

    def body(j, carry):
        pltpu.sync_copy(onesv, acc.at[cv.at[j]], add=True)
        return carry

    lax.fori_loop(0, CPT, body, 0)
    plsc.subcore_barrier()
    pltpu.sync_copy(
        acc.at[pl.ds(s * RPT, RPT)], out.at[c, pl.ds(s * RPT, RPT)]
    )


@functools.partial(
    pl.kernel,
    mesh=_MESH,
    out_type=jax.ShapeDtypeStruct((NCORES, NPAD, DH), jnp.float32),
    scratch_types=[
        pltpu.VMEM((CPT, CHUNK), jnp.int32),
        pltpu.VMEM((CPT, CHUNK), jnp.int32),
        [pltpu.VMEM((CHUNK, DH), jnp.float32) for _ in range(GRP)],
        [pltpu.SemaphoreType.DMA for _ in range(GRP)],
        pltpu.VMEM_SHARED((NPAD, DH), jnp.float32),
        pltpu.VMEM_SHARED((NPAD, DH), jnp.float32),
    ],
    compiler_params=pltpu.CompilerParams(use_tc_tiling_on_sc=False),
)
def _prop_sc(ys, rowi, coli, zrows, out, rv, cv, gbufs, sems, acc, ys_sh):
    c = lax.axis_index("c")
    s = lax.axis_index("s")
    w = c * NSUB + s
    pltpu.sync_copy(zrows.at[pl.ds(s * RPT, RPT)], acc.at[pl.ds(s * RPT, RPT)])
    pltpu.sync_copy(ys.at[pl.ds(s * RPT, RPT)], ys_sh.at[pl.ds(s * RPT, RPT)])
    pltpu.sync_copy(rowi.at[w], rv)
    pltpu.sync_copy(coli.at[w], cv)
    plsc.subcore_barrier()

    def body(i, carry):
        base = i * GRP
        cps = [
            pltpu.async_copy(ys_sh.at[rv.at[base + b]], gbufs[b], sems[b])
            for b in range(GRP)
        ]
        for b in range(GRP):
            cps[b].wait()
            pltpu.sync_copy(gbufs[b], acc.at[cv.at[base + b]], add=True)
        return carry

    lax.fori_loop(0, CPT // GRP, body, 0)
    plsc.subcore_barrier()
    pltpu.sync_copy(
        acc.at[pl.ds(s * RPT, RPT)], out.at[c, pl.ds(s * RPT, RPT)]
    )


def _tc1_body(x_ref, wc_ref, dega_ref, xw0_ref, ys1_ref, dis_ref):
    deg = dega_ref[0] + dega_ref[1]
    dis = jnp.where(deg > 0.0, lax.rsqrt(deg), 0.0)
    dis_ref[...] = dis
    xw = jnp.dot(x_ref[...], wc_ref[...], preferred_element_type=jnp.float32)
    xw0_ref[...] = xw[:, :DH]
    ys1_ref[0:N, :] = dis[0:N, :] * xw[:, DH:]
    ys1_ref[N:NPAD, :] = jnp.zeros((NPAD - N, DH), jnp.float32)


_tc1 = pl.pallas_call(
    _tc1_body,
    out_shape=(
        jax.ShapeDtypeStruct((N, DH), jnp.float32),
        jax.ShapeDtypeStruct((NPAD, DH), jnp.float32),
        jax.ShapeDtypeStruct((NPAD, DH), jnp.float32),
    ),
)


def _tc2_body(xw0_ref, p1a_ref, dis_ref, b1_ref, w2c_ref, hw0_ref, ys2_ref):
    p1 = (p1a_ref[0, 0:N, :] + p1a_ref[1, 0:N, :]) * dis_ref[0:N, :]
    h = jnp.maximum(xw0_ref[...] + p1 + b1_ref[...], 0.0)
    hw = jnp.dot(h, w2c_ref[...], preferred_element_type=jnp.float32)
    hw0_ref[...] = hw[:, :DH]
    ys2_ref[0:N, :] = dis_ref[0:N, :] * hw[:, DH:]
    ys2_ref[N:NPAD, :] = jnp.zeros((NPAD - N, DH), jnp.float32)


_tc2 = pl.pallas_call(
    _tc2_body,
    out_shape=(
        jax.ShapeDtypeStruct((N, DH), jnp.float32),
        jax.ShapeDtypeStruct((NPAD, DH), jnp.float32),
    ),
)


def _tc3_body(hw0_ref, p2a_ref, dis_ref, b2_ref, out_ref):
    p2 = (p2a_ref[0, 0:N, :] + p2a_ref[1, 0:N, :]) * dis_ref[0:N, :]
    o = hw0_ref[...] + p2 + b2_ref[...]
    z = o - jnp.max(o, axis=1, keepdims=True)
    lse = jnp.log(jnp.sum(jnp.exp(z), axis=1, keepdims=True))
    out_ref[...] = z - lse


_tc3 = pl.pallas_call(
    _tc3_body,
    out_shape=jax.ShapeDtypeStruct((N, DH), jnp.float32),
)


def kernel(x, edge_index, W1_0, W1_1, b1, W2_0, W2_1, b2):
    ei = edge_index.astype(jnp.int32)
    row, col = ei[0], ei[1]
    rowp = jnp.concatenate(
        [row, jnp.zeros((EPAD - E,), jnp.int32)]
    ).reshape(NW, CPT, CHUNK)
    colp = jnp.concatenate(
        [col, jnp.full((EPAD - E,), N, jnp.int32)]
    ).reshape(NW, CPT, CHUNK)
    zrows = jnp.zeros((NPAD, DH), jnp.float32)
    onesb = jnp.ones((CHUNK, DH), jnp.float32)
    wc1 = jnp.concatenate([W1_0, W1_1], axis=1)
    wc2 = jnp.concatenate([W2_0, W2_1], axis=1)

    dega = _deg_sc(colp, onesb, zrows)
    xw0, ys1, dis = _tc1(x, wc1, dega)
    p1a = _prop_sc(ys1, rowp, colp, zrows)
    hw0, ys2 = _tc2(xw0, p1a, dis, b1.reshape(1, DH), wc2)
    p2a = _prop_sc(ys2, rowp, colp, zrows)
    return _tc3(hw0, p2a, dis, b2.reshape(1, DH))
